# baseline (speedup 1.0000x reference)
import functools

import jax
import jax.numpy as jnp
from jax import lax
from jax.experimental import pallas as pl
from jax.experimental.pallas import tpu as pltpu
from jax.experimental.pallas import tpu_sc as plsc

_N = 1024
_LOWER = -4.0
_UPPER = 4.0
_DT = 0.01
_B = 131072

_NC = 2
_NS = 16
_NW = _NC * _NS
_L = 16
_NB = _B // 128
_BLOCKS = _NB // _NW
_VPB = 128 // _L

_SCALE = _N / (_UPPER - _LOWER)
_BIAS = 0.5 - _LOWER * _SCALE


@functools.partial(
    pl.kernel,
    out_type=jax.ShapeDtypeStruct((_NB, 2, 128), jnp.float32),
    mesh=plsc.VectorSubcoreMesh(core_axis_name="c", subcore_axis_name="s"),
    scratch_types=[
        pltpu.VMEM((_BLOCKS, 2, 128), jnp.float32),
        pltpu.VMEM((_N,), jnp.float32),
        pltpu.SemaphoreType.DMA,
    ],
    compiler_params=pltpu.CompilerParams(
        needs_layout_passes=False, use_tc_tiling_on_sc=False
    ),
)
def _sc_kernel(x_hbm, force_hbm, out_hbm, buf, force_v, sem):
    wid = lax.axis_index("s") * _NC + lax.axis_index("c")
    base = wid * _BLOCKS
    half = _BLOCKS // 2
    in0 = pltpu.async_copy(x_hbm.at[pl.ds(base, half)], buf.at[pl.ds(0, half)], sem)
    in1 = pltpu.async_copy(
        x_hbm.at[pl.ds(base + half, half)], buf.at[pl.ds(half, half)], sem
    )
    pltpu.sync_copy(force_hbm, force_v)

    @plsc.parallel_loop(0, _N // _L, unroll=8)
    def prescale(i):
        sl = pl.ds(i * _L, _L)
        force_v[sl] = force_v[sl] * _DT

    def make_body(block_off):
        def body(i):
            j = block_off + i // _VPB
            sl = pl.ds((i % _VPB) * _L, _L)
            u = buf[j, 0, sl] * _SCALE + _BIAS
            u = jnp.minimum(jnp.maximum(u, 0.5), float(_N - 1) + 0.5)
            idx = u.astype(jnp.int32)
            buf[j, 1, sl] = buf[j, 1, sl] + plsc.load_gather(force_v, [idx])
        return body

    in0.wait()
    plsc.parallel_loop(0, half * _VPB, unroll=8)(make_body(0))
    out0 = pltpu.async_copy(buf.at[pl.ds(0, half)], out_hbm.at[pl.ds(base, half)], sem)
    in1.wait()
    plsc.parallel_loop(0, half * _VPB, unroll=8)(make_body(half))
    out1 = pltpu.async_copy(
        buf.at[pl.ds(half, half)], out_hbm.at[pl.ds(base + half, half)], sem
    )
    out0.wait()
    out1.wait()


def kernel(X, force):
    xb = X.reshape(_NB, 128, 2).transpose(0, 2, 1)
    ob = _sc_kernel(xb, force)
    return ob.transpose(0, 2, 1).reshape(_B, 2)


# speedup vs baseline: 1.0029x; 1.0029x over previous
import functools

import jax
import jax.numpy as jnp
from jax import lax
from jax.experimental import pallas as pl
from jax.experimental.pallas import tpu as pltpu
from jax.experimental.pallas import tpu_sc as plsc

_N = 1024
_LOWER = -4.0
_UPPER = 4.0
_DT = 0.01
_B = 131072

_NC = 2
_NS = 16
_NW = _NC * _NS
_L = 16
_NB = _B // 128
_BLOCKS = _NB // _NW
_VPB = 128 // _L

_SCALE = _N / (_UPPER - _LOWER)
_BIAS = 0.5 - _LOWER * _SCALE


@functools.partial(
    pl.kernel,
    out_type=jax.ShapeDtypeStruct((_NB, 2, 128), jnp.float32),
    mesh=plsc.VectorSubcoreMesh(core_axis_name="c", subcore_axis_name="s"),
    scratch_types=[
        pltpu.VMEM((_BLOCKS, 2, 128), jnp.float32),
        pltpu.VMEM((_N,), jnp.float32),
        pltpu.SemaphoreType.DMA,
        pltpu.SemaphoreType.DMA,
        pltpu.SemaphoreType.DMA,
    ],
    compiler_params=pltpu.CompilerParams(
        needs_layout_passes=False, use_tc_tiling_on_sc=False
    ),
)
def _sc_kernel(x_hbm, force_hbm, out_hbm, buf, force_v, sem, sem1, sem2):
    wid = lax.axis_index("s") * _NC + lax.axis_index("c")
    base = wid * _BLOCKS
    half = _BLOCKS // 2
    in0 = pltpu.async_copy(x_hbm.at[pl.ds(base, half)], buf.at[pl.ds(0, half)], sem)
    in1 = pltpu.async_copy(
        x_hbm.at[pl.ds(base + half, half)], buf.at[pl.ds(half, half)], sem1
    )
    pltpu.sync_copy(force_hbm, force_v)

    @plsc.parallel_loop(0, _N // _L, unroll=8)
    def prescale(i):
        sl = pl.ds(i * _L, _L)
        force_v[sl] = force_v[sl] * _DT

    def make_body(block_off):
        def body(i):
            j = block_off + i // _VPB
            sl = pl.ds((i % _VPB) * _L, _L)
            u = buf[j, 0, sl] * _SCALE + _BIAS
            u = jnp.minimum(jnp.maximum(u, 0.5), float(_N - 1) + 0.5)
            idx = u.astype(jnp.int32)
            buf[j, 1, sl] = buf[j, 1, sl] + plsc.load_gather(force_v, [idx])
        return body

    in0.wait()
    plsc.parallel_loop(0, half * _VPB, unroll=8)(make_body(0))
    out0 = pltpu.async_copy(buf.at[pl.ds(0, half)], out_hbm.at[pl.ds(base, half)], sem2)
    in1.wait()
    plsc.parallel_loop(0, half * _VPB, unroll=8)(make_body(half))
    out1 = pltpu.async_copy(
        buf.at[pl.ds(half, half)], out_hbm.at[pl.ds(base + half, half)], sem2
    )
    out0.wait()
    out1.wait()


def kernel(X, force):
    xb = X.reshape(_NB, 128, 2).transpose(0, 2, 1)
    ob = _sc_kernel(xb, force)
    return ob.transpose(0, 2, 1).reshape(_B, 2)


# speedup vs baseline: 1.0254x; 1.0224x over previous
import functools

import jax
import jax.numpy as jnp
from jax import lax
from jax.experimental import pallas as pl
from jax.experimental.pallas import tpu as pltpu
from jax.experimental.pallas import tpu_sc as plsc

_N = 1024
_LOWER = -4.0
_UPPER = 4.0
_DT = 0.01
_B = 131072

_NC = 2
_NS = 16
_NW = _NC * _NS
_L = 16
_NB = _B // 128
_BLOCKS = _NB // _NW
_VPB = 128 // _L

_SCALE = _N / (_UPPER - _LOWER)
_BIAS = 0.5 - _LOWER * _SCALE


@functools.partial(
    pl.kernel,
    out_type=jax.ShapeDtypeStruct((_NB, 2, 128), jnp.float32),
    mesh=plsc.VectorSubcoreMesh(core_axis_name="c", subcore_axis_name="s"),
    scratch_types=[
        pltpu.VMEM((_BLOCKS, 2, 128), jnp.float32),
        pltpu.VMEM((_N,), jnp.float32),
        pltpu.SemaphoreType.DMA,
        pltpu.SemaphoreType.DMA,
        pltpu.SemaphoreType.DMA,
    ],
    compiler_params=pltpu.CompilerParams(
        needs_layout_passes=False, use_tc_tiling_on_sc=False
    ),
)
def _sc_kernel(x_hbm, force_hbm, out_hbm, buf, force_v, sem, sem1, sem2):
    wid = lax.axis_index("s") * _NC + lax.axis_index("c")
    base = wid * _BLOCKS
    slab = pltpu.async_copy(x_hbm.at[pl.ds(base, _BLOCKS)], buf, sem)
    pltpu.sync_copy(force_hbm, force_v)

    @plsc.parallel_loop(0, _N // _L, unroll=8)
    def prescale(i):
        sl = pl.ds(i * _L, _L)
        force_v[sl] = force_v[sl] * _DT

    slab.wait()

    @plsc.parallel_loop(0, _BLOCKS * _VPB, unroll=16)
    def step(i):
        j = i // _VPB
        sl = pl.ds((i % _VPB) * _L, _L)
        u = buf[j, 0, sl] * _SCALE + _BIAS
        u = jnp.minimum(jnp.maximum(u, 0.5), float(_N - 1) + 0.5)
        idx = u.astype(jnp.int32)
        buf[j, 1, sl] = buf[j, 1, sl] + plsc.load_gather(force_v, [idx])

    pltpu.sync_copy(buf, out_hbm.at[pl.ds(base, _BLOCKS)])


def kernel(X, force):
    xb = X.reshape(_NB, 128, 2).transpose(0, 2, 1)
    ob = _sc_kernel(xb, force)
    return ob.transpose(0, 2, 1).reshape(_B, 2)
